# baseline (speedup 1.0000x reference)
import functools

import jax
import jax.numpy as jnp
from jax import lax
from jax.experimental import pallas as pl
from jax.experimental.pallas import tpu as pltpu
from jax.experimental.pallas import tpu_sc as plsc

_FAN_IN = 1024
_FAN_OUT = 1024
_K = 2048
_NW = 32
_JP = 1032
_NCB = _FAN_OUT // 128
_TPW = 32
_G_TOTAL = _NCB * _JP * 128


def _gather_body(w_hbm, ht_hbm, g_hbm, w_v, h_v, g_v, hx_v, gx_v):
    wid = lax.axis_index("s") * 2 + lax.axis_index("c")
    pltpu.sync_copy(w_hbm, w_v)
    pltpu.sync_copy(ht_hbm.at[pl.ds(_TPW * wid, _TPW)], h_v)

    @plsc.parallel_loop(0, _TPW // 8, 1)
    def _(rb):
        for sr in range(8):
            x = rb * 8 + sr
            for cb in range(_NCB):
                for v in range(8):
                    idx = h_v[x, pl.ds(cb * 128 + 16 * v, 16)]
                    val = plsc.load_gather(w_v, [idx])
                    g_v[pl.ds(rb * 1024 + (cb * _TPW + sr) * 128 + 16 * v,
                              16)] = val

    for cb in range(_NCB):
        blk = _TPW * 128
        pltpu.sync_copy(
            g_v.at[pl.ds(cb * blk, blk)],
            g_hbm.at[pl.ds((cb * _JP + _TPW * wid) * 128, blk)],
        )

    @pl.when(wid < _NCB)
    def _():
        pltpu.sync_copy(ht_hbm.at[pl.ds(1024 + wid, 1)], hx_v)
        for cb in range(_NCB):
            for v in range(8):
                idx = hx_v[0, pl.ds(cb * 128 + 16 * v, 16)]
                val = plsc.load_gather(w_v, [idx])
                gx_v[pl.ds(cb * 128 + 16 * v, 16)] = val
        for cb in range(_NCB):
            pltpu.sync_copy(
                gx_v.at[pl.ds(cb * 128, 128)],
                g_hbm.at[pl.ds((cb * _JP + 1024 + wid) * 128, 128)],
            )


_gather = functools.partial(
    pl.kernel,
    mesh=plsc.VectorSubcoreMesh(core_axis_name="c", subcore_axis_name="s"),
    out_type=jax.ShapeDtypeStruct((_G_TOTAL,), jnp.float32),
    scratch_types=[
        pltpu.VMEM((_K,), jnp.float32),
        pltpu.VMEM((_TPW, 1024), jnp.int32),
        pltpu.VMEM((_TPW * 1024,), jnp.float32),
        pltpu.VMEM((1, 1024), jnp.int32),
        pltpu.VMEM((1024,), jnp.float32),
    ],
    compiler_params=pltpu.CompilerParams(needs_layout_passes=False),
)(_gather_body)


def _matmul_body(a_ref, g_ref, o_ref):
    a = a_ref[...]
    b = a.shape[0]
    tail = (lax.broadcasted_iota(jnp.int32, (b, _JP - _FAN_IN), 1) == 0)
    a_aug = jnp.concatenate([a, tail.astype(jnp.float32)], axis=1)
    for c in range(_NCB):
        o_ref[:, c * 128:(c + 1) * 128] = lax.dot_general(
            a_aug,
            g_ref[pl.ds(c * _JP, _JP), :],
            (((1,), (0,)), ((), ())),
            preferred_element_type=jnp.float32,
        )


def kernel(a, W, H):
    ht = jnp.pad(H.T, ((0, _JP - (_FAN_IN + 1)), (0, 0)))
    g = _gather(W, ht)
    g2 = g.reshape(_NCB * _JP, 128)
    return pl.pallas_call(
        _matmul_body,
        out_shape=jax.ShapeDtypeStruct((a.shape[0], _FAN_OUT), jnp.float32),
    )(a, g2)


# speedup vs baseline: 1.0210x; 1.0210x over previous
import functools

import jax
import jax.numpy as jnp
from jax import lax
from jax.experimental import pallas as pl
from jax.experimental.pallas import tpu as pltpu
from jax.experimental.pallas import tpu_sc as plsc

_FAN_IN = 1024
_FAN_OUT = 1024
_K = 2048
_NW = 32
_NJ = _FAN_IN + 1
_NCB = _FAN_OUT // 128
_JPW = 32
_PER_W = _JPW * _FAN_OUT
_G_TOTAL = _NCB * _NJ * 128


def _gather_body(w_hbm, h_hbm, g_hbm, w_v, h_v, g_v, hx_v, gx_v):
    wid = lax.axis_index("s") * 2 + lax.axis_index("c")
    pltpu.sync_copy(w_hbm, w_v)
    pltpu.sync_copy(h_hbm.at[pl.ds(_PER_W * wid, _PER_W)], h_v)

    @plsc.parallel_loop(0, _JPW // 8, 1)
    def _(rb):
        for sr in range(8):
            x = rb * 8 + sr
            for cb in range(_NCB):
                for v in range(8):
                    idx = h_v[pl.ds(x * 1024 + cb * 128 + 16 * v, 16)]
                    val = plsc.load_gather(w_v, [idx])
                    g_v[pl.ds(rb * 1024 + (cb * _JPW + sr) * 128 + 16 * v,
                              16)] = val

    for cb in range(_NCB):
        blk = _JPW * 128
        pltpu.sync_copy(
            g_v.at[pl.ds(cb * blk, blk)],
            g_hbm.at[pl.ds((cb * _NJ + _JPW * wid) * 128, blk)],
        )

    @pl.when(wid == _NW - 1)
    def _():
        pltpu.sync_copy(h_hbm.at[pl.ds(1024 * 1024, 1024)], hx_v)
        for cb in range(_NCB):
            for v in range(8):
                idx = hx_v[pl.ds(cb * 128 + 16 * v, 16)]
                val = plsc.load_gather(w_v, [idx])
                gx_v[pl.ds(cb * 128 + 16 * v, 16)] = val
        for cb in range(_NCB):
            pltpu.sync_copy(
                gx_v.at[pl.ds(cb * 128, 128)],
                g_hbm.at[pl.ds((cb * _NJ + 1024) * 128, 128)],
            )


_gather = functools.partial(
    pl.kernel,
    mesh=plsc.VectorSubcoreMesh(core_axis_name="c", subcore_axis_name="s"),
    out_type=jax.ShapeDtypeStruct((_G_TOTAL,), jnp.float32),
    scratch_types=[
        pltpu.VMEM((_K,), jnp.float32),
        pltpu.VMEM((_PER_W,), jnp.int32),
        pltpu.VMEM((_JPW * 1024,), jnp.float32),
        pltpu.VMEM((1024,), jnp.int32),
        pltpu.VMEM((1024,), jnp.float32),
    ],
    compiler_params=pltpu.CompilerParams(needs_layout_passes=False),
)(_gather_body)


def _matmul_body(a_ref, g_ref, o_ref):
    a = a_ref[...]
    b = a.shape[0]
    a_aug = jnp.concatenate([a, jnp.ones((b, 1), jnp.float32)], axis=1)
    for c in range(_NCB):
        o_ref[:, c * 128:(c + 1) * 128] = lax.dot_general(
            a_aug,
            g_ref[pl.ds(c * _NJ, _NJ), :],
            (((1,), (0,)), ((), ())),
            preferred_element_type=jnp.float32,
        )


def kernel(a, W, H):
    g = _gather(W, H.T.reshape(-1))
    g2 = g.reshape(_NCB * _NJ, 128)
    return pl.pallas_call(
        _matmul_body,
        out_shape=jax.ShapeDtypeStruct((a.shape[0], _FAN_OUT), jnp.float32),
    )(a, g2)


# speedup vs baseline: 1.4448x; 1.4150x over previous
import functools

import jax
import jax.numpy as jnp
from jax import lax
from jax.experimental import pallas as pl
from jax.experimental.pallas import tpu as pltpu
from jax.experimental.pallas import tpu_sc as plsc

_FAN_IN = 1024
_FAN_OUT = 1024
_K = 2048
_NW = 32
_NJ = _FAN_IN + 1
_NCB = _FAN_OUT // 128
_JPW = 32
_PER_W = _JPW * _FAN_OUT
_G_TOTAL = _NCB * _NJ * 128


def _gather_body(w_hbm, h_hbm, g_hbm, w_v, h_v, g_v, hx_v, gx_v):
    wid = lax.axis_index("s") * 2 + lax.axis_index("c")
    pltpu.sync_copy(w_hbm, w_v)
    pltpu.sync_copy(h_hbm.at[pl.ds(_PER_W * wid, _PER_W)], h_v)

    @plsc.parallel_loop(0, _JPW, 1, unroll=2)
    def _(x):
        hbase = x * 1024
        gbase = x * 128
        for cb in range(_NCB):
            for v in range(8):
                idx = h_v[pl.ds(hbase + cb * 128 + 16 * v, 16)]
                val = plsc.load_gather(w_v, [idx])
                g_v[pl.ds(cb * _JPW * 128 + gbase + 16 * v, 16)] = val

    for cb in range(_NCB):
        blk = _JPW * 128
        pltpu.sync_copy(
            g_v.at[pl.ds(cb * blk, blk)],
            g_hbm.at[pl.ds((cb * _NJ + _JPW * wid) * 128, blk)],
        )

    @pl.when(wid == _NW - 1)
    def _():
        pltpu.sync_copy(h_hbm.at[pl.ds(1024 * 1024, 1024)], hx_v)
        for cb in range(_NCB):
            for v in range(8):
                idx = hx_v[pl.ds(cb * 128 + 16 * v, 16)]
                val = plsc.load_gather(w_v, [idx])
                gx_v[pl.ds(cb * 128 + 16 * v, 16)] = val
        for cb in range(_NCB):
            pltpu.sync_copy(
                gx_v.at[pl.ds(cb * 128, 128)],
                g_hbm.at[pl.ds((cb * _NJ + 1024) * 128, 128)],
            )


_gather = functools.partial(
    pl.kernel,
    mesh=plsc.VectorSubcoreMesh(core_axis_name="c", subcore_axis_name="s"),
    out_type=jax.ShapeDtypeStruct((_G_TOTAL,), jnp.float32),
    scratch_types=[
        pltpu.VMEM((_K,), jnp.float32),
        pltpu.VMEM((_PER_W,), jnp.int32),
        pltpu.VMEM((_JPW * 1024,), jnp.float32),
        pltpu.VMEM((1024,), jnp.int32),
        pltpu.VMEM((1024,), jnp.float32),
    ],
    compiler_params=pltpu.CompilerParams(needs_layout_passes=False),
)(_gather_body)


def _matmul_body(a_ref, g_ref, o_ref):
    a = a_ref[...]
    b = a.shape[0]
    a_aug = jnp.concatenate([a, jnp.ones((b, 1), jnp.float32)], axis=1)
    for c in range(_NCB):
        o_ref[:, c * 128:(c + 1) * 128] = lax.dot_general(
            a_aug,
            g_ref[pl.ds(c * _NJ, _NJ), :],
            (((1,), (0,)), ((), ())),
            preferred_element_type=jnp.float32,
        )


def kernel(a, W, H):
    g = _gather(W, H.T.reshape(-1))
    g2 = g.reshape(_NCB * _NJ, 128)
    return pl.pallas_call(
        _matmul_body,
        out_shape=jax.ShapeDtypeStruct((a.shape[0], _FAN_OUT), jnp.float32),
    )(a, g2)
